# baseline (speedup 1.0000x reference)
import functools

import jax
import jax.numpy as jnp
from jax import lax
from jax.experimental import pallas as pl
from jax.experimental.pallas import tpu as pltpu
from jax.experimental.pallas import tpu_sc as plsc

Q = 320000
D = 128
C = 10
CP = 16
NC = 2
NS = 16
NW = NC * NS
ROWS_PER_W = Q // NW
P = 250
NP = ROWS_PER_W // P
NEG_INF = float("-inf")


def _sc_segmax_body(
    q_hbm, starts_hbm, ends_hbm, out_hbm, buf0, buf1, acc, sv, ev, sem0, sem1
):
    wid = lax.axis_index("s") * NC + lax.axis_index("c")
    lo = wid * ROWS_PER_W

    pltpu.sync_copy(starts_hbm, sv)
    pltpu.sync_copy(ends_hbm, ev)

    neg = jnp.full((16,), NEG_INF, jnp.float32)

    def init_body(i, _):
        acc[pl.ds(i * 16, 16)] = neg
        return 0

    lax.fori_loop(0, (CP * D) // 16, init_body, 0)

    svv = sv[pl.ds(0, 16)]
    evv = ev[pl.ds(0, 16)]
    starts_s = [svv[c] for c in range(C)]
    ends_s = [evv[c] for c in range(C)]

    def start(p, buf, sem):
        pltpu.async_copy(q_hbm.at[pl.ds((lo + p * P) * D, P * D)], buf, sem)

    def wait(buf, sem):
        pltpu.make_async_copy(q_hbm.at[pl.ds(0, P * D)], buf, sem).wait()

    def process(p, buf):
        row0 = lo + p * P
        for c in range(C):
            rs = jnp.clip(starts_s[c] - row0, 0, P)
            re = jnp.clip(ends_s[c] - row0, 0, P)
            n4 = (re - rs) // 4
            a = tuple(acc[pl.ds(c * D + dc * 16, 16)] for dc in range(8))

            def rbody4(k, a):
                base = (rs + k * 4) * D
                out = []
                for dc in range(8):
                    off = base + dc * 16
                    m01 = jnp.maximum(
                        buf[pl.ds(off, 16)], buf[pl.ds(off + D, 16)]
                    )
                    m23 = jnp.maximum(
                        buf[pl.ds(off + 2 * D, 16)], buf[pl.ds(off + 3 * D, 16)]
                    )
                    out.append(jnp.maximum(a[dc], jnp.maximum(m01, m23)))
                return tuple(out)

            a = lax.fori_loop(0, n4, rbody4, a)

            def rbody(r, a):
                base = r * D
                return tuple(
                    jnp.maximum(a[dc], buf[pl.ds(base + dc * 16, 16)])
                    for dc in range(8)
                )

            a = lax.fori_loop(rs + n4 * 4, re, rbody, a)
            for dc in range(8):
                acc[pl.ds(c * D + dc * 16, 16)] = a[dc]

    start(0, buf0, sem0)
    start(1, buf1, sem1)

    def gbody(g, _):
        p0 = 2 * g
        wait(buf0, sem0)
        process(p0, buf0)

        @pl.when(p0 + 2 < NP)
        def _():
            start(p0 + 2, buf0, sem0)

        wait(buf1, sem1)
        process(p0 + 1, buf1)

        @pl.when(p0 + 3 < NP)
        def _():
            start(p0 + 3, buf1, sem1)

        return 0

    lax.fori_loop(0, NP // 2, gbody, 0)
    pltpu.sync_copy(acc, out_hbm.at[pl.ds(wid * CP * D, CP * D)])


_sc_segmax = functools.partial(
    pl.kernel,
    out_type=jax.ShapeDtypeStruct((NW * CP * D,), jnp.float32),
    mesh=plsc.VectorSubcoreMesh(core_axis_name="c", subcore_axis_name="s"),
    scratch_types=[
        pltpu.VMEM((P * D,), jnp.float32),
        pltpu.VMEM((P * D,), jnp.float32),
        pltpu.VMEM((CP * D,), jnp.float32),
        pltpu.VMEM((16,), jnp.int32),
        pltpu.VMEM((16,), jnp.int32),
        pltpu.SemaphoreType.DMA,
        pltpu.SemaphoreType.DMA,
    ],
)(_sc_segmax_body)


def _tc_gcn_body(part_ref, counts_ref, pad_ref, w_ref, b_ref, cc_ref, out_ref):
    seg = part_ref[0]
    for i in range(1, NW):
        seg = jnp.maximum(seg, part_ref[i])
    has = counts_ref[:] > 0
    core = jnp.where(has, seg, pad_ref[:])

    cc = cc_ref[:]
    rr = lax.broadcasted_iota(jnp.int32, (CP, CP), 0)
    cidx = lax.broadcasted_iota(jnp.int32, (CP, CP), 1)
    eye = jnp.where(rr == cidx, 1.0, 0.0).astype(jnp.float32)
    a_hat = cc + eye
    deg = jnp.sum(a_hat, axis=1, keepdims=True)
    dinv = jnp.where(deg > 0, lax.rsqrt(deg), 0.0)
    t = jnp.dot(a_hat, core * dinv, preferred_element_type=jnp.float32)
    h = jnp.dot(t * dinv, w_ref[:], preferred_element_type=jnp.float32)
    out_ref[:] = jnp.maximum(h + b_ref[:], 0.0)


_tc_gcn = pl.pallas_call(
    _tc_gcn_body,
    out_shape=jax.ShapeDtypeStruct((CP, D), jnp.float32),
)


def kernel(prev_assign, q_embeddings, padding_emb, W, b, core_con):
    pa = prev_assign.astype(jnp.int32)
    starts = jnp.sum(
        (pa[:, None] < jnp.arange(C + 1, dtype=jnp.int32)[None, :]).astype(jnp.int32),
        axis=0,
    )
    pad6 = jnp.full((CP - C,), Q, jnp.int32)
    starts16 = jnp.concatenate([starts[:C], pad6])
    ends16 = jnp.concatenate([starts[1:], pad6])
    counts16 = jnp.concatenate(
        [starts[1:] - starts[:C], jnp.zeros((CP - C,), jnp.int32)]
    ).reshape(CP, 1)

    partials = _sc_segmax(q_embeddings.reshape(-1), starts16, ends16)
    partials = partials.reshape(NW, CP, D)

    cc16 = jnp.zeros((CP, CP), jnp.float32).at[:C, :C].set(core_con)
    out16 = _tc_gcn(
        partials,
        counts16,
        padding_emb.reshape(1, D),
        W,
        b.reshape(1, D),
        cc16,
    )
    return out16[:C]


# speedup vs baseline: 1.7503x; 1.7503x over previous
import functools

import jax
import jax.numpy as jnp
from jax import lax
from jax.experimental import pallas as pl
from jax.experimental.pallas import tpu as pltpu
from jax.experimental.pallas import tpu_sc as plsc

Q = 320000
D = 128
C = 10
CP = 16
NC = 2
NS = 16
NW = NC * NS
ROWS_PER_W = Q // NW
P = 500
NP = ROWS_PER_W // P
NEG_INF = float("-inf")


def _sc_segmax_body(
    q_hbm, starts_hbm, ends_hbm, out_hbm, buf0, buf1, acc, sv, ev, sem0, sem1
):
    wid = lax.axis_index("s") * NC + lax.axis_index("c")
    lo = wid * ROWS_PER_W

    pltpu.sync_copy(starts_hbm, sv)
    pltpu.sync_copy(ends_hbm, ev)

    neg = jnp.full((16,), NEG_INF, jnp.float32)

    def init_body(i, _):
        acc[pl.ds(i * 16, 16)] = neg
        return 0

    lax.fori_loop(0, (CP * D) // 16, init_body, 0)

    svv = sv[pl.ds(0, 16)]
    evv = ev[pl.ds(0, 16)]
    starts_s = [svv[c] for c in range(C)]
    ends_s = [evv[c] for c in range(C)]

    def start(p, buf, sem):
        pltpu.async_copy(q_hbm.at[pl.ds((lo + p * P) * D, P * D)], buf, sem)

    def wait(buf, sem):
        pltpu.make_async_copy(q_hbm.at[pl.ds(0, P * D)], buf, sem).wait()

    def process(p, buf):
        row0 = lo + p * P
        for c in range(C):
            rs = jnp.clip(starts_s[c] - row0, 0, P)
            re = jnp.clip(ends_s[c] - row0, 0, P)
            a = tuple(acc[pl.ds(c * D + dc * 16, 16)] for dc in range(8))

            def rbody(r, a):
                base = r * D
                return tuple(
                    jnp.maximum(a[dc], buf[pl.ds(base + dc * 16, 16)])
                    for dc in range(8)
                )

            a = lax.fori_loop(rs, re, rbody, a)
            for dc in range(8):
                acc[pl.ds(c * D + dc * 16, 16)] = a[dc]

    start(0, buf0, sem0)
    start(1, buf1, sem1)

    def gbody(g, _):
        p0 = 2 * g
        wait(buf0, sem0)
        process(p0, buf0)

        @pl.when(p0 + 2 < NP)
        def _():
            start(p0 + 2, buf0, sem0)

        wait(buf1, sem1)
        process(p0 + 1, buf1)

        @pl.when(p0 + 3 < NP)
        def _():
            start(p0 + 3, buf1, sem1)

        return 0

    lax.fori_loop(0, NP // 2, gbody, 0)
    pltpu.sync_copy(acc, out_hbm.at[pl.ds(wid * CP * D, CP * D)])


_sc_segmax = functools.partial(
    pl.kernel,
    out_type=jax.ShapeDtypeStruct((NW * CP * D,), jnp.float32),
    mesh=plsc.VectorSubcoreMesh(core_axis_name="c", subcore_axis_name="s"),
    scratch_types=[
        pltpu.VMEM((P * D,), jnp.float32),
        pltpu.VMEM((P * D,), jnp.float32),
        pltpu.VMEM((CP * D,), jnp.float32),
        pltpu.VMEM((16,), jnp.int32),
        pltpu.VMEM((16,), jnp.int32),
        pltpu.SemaphoreType.DMA,
        pltpu.SemaphoreType.DMA,
    ],
)(_sc_segmax_body)


def _tc_gcn_body(part_ref, counts_ref, pad_ref, w_ref, b_ref, cc_ref, out_ref):
    seg = part_ref[0]
    for i in range(1, NW):
        seg = jnp.maximum(seg, part_ref[i])
    has = counts_ref[:] > 0
    core = jnp.where(has, seg, pad_ref[:])

    cc = cc_ref[:]
    rr = lax.broadcasted_iota(jnp.int32, (CP, CP), 0)
    cidx = lax.broadcasted_iota(jnp.int32, (CP, CP), 1)
    eye = jnp.where(rr == cidx, 1.0, 0.0).astype(jnp.float32)
    a_hat = cc + eye
    deg = jnp.sum(a_hat, axis=1, keepdims=True)
    dinv = jnp.where(deg > 0, lax.rsqrt(deg), 0.0)
    t = jnp.dot(a_hat, core * dinv, preferred_element_type=jnp.float32)
    h = jnp.dot(t * dinv, w_ref[:], preferred_element_type=jnp.float32)
    out_ref[:] = jnp.maximum(h + b_ref[:], 0.0)


_tc_gcn = pl.pallas_call(
    _tc_gcn_body,
    out_shape=jax.ShapeDtypeStruct((CP, D), jnp.float32),
)


def kernel(prev_assign, q_embeddings, padding_emb, W, b, core_con):
    pa = prev_assign.astype(jnp.int32)
    starts = jnp.sum(
        (pa[:, None] < jnp.arange(C + 1, dtype=jnp.int32)[None, :]).astype(jnp.int32),
        axis=0,
    )
    pad6 = jnp.full((CP - C,), Q, jnp.int32)
    starts16 = jnp.concatenate([starts[:C], pad6])
    ends16 = jnp.concatenate([starts[1:], pad6])
    counts16 = jnp.concatenate(
        [starts[1:] - starts[:C], jnp.zeros((CP - C,), jnp.int32)]
    ).reshape(CP, 1)

    partials = _sc_segmax(q_embeddings.reshape(-1), starts16, ends16)
    partials = partials.reshape(NW, CP, D)

    cc16 = jnp.zeros((CP, CP), jnp.float32).at[:C, :C].set(core_con)
    out16 = _tc_gcn(
        partials,
        counts16,
        padding_emb.reshape(1, D),
        W,
        b.reshape(1, D),
        cc16,
    )
    return out16[:C]
